# baseline (speedup 1.0000x reference)
import functools

import jax
import jax.numpy as jnp
from jax import lax
from jax.experimental import pallas as pl
from jax.experimental.pallas import tpu as pltpu
from jax.experimental.pallas import tpu_sc as plsc

B = 8
MAX_LEN = 2048
TOTAL = 8192
N_FEAT = 4
CARD_P2 = 1026
D = 256

NC = 2
NS = 16
L = 16
NW = NC * NS
TOK_PER_W = TOTAL // NW
CHUNK = 32
N_CHUNK = TOK_PER_W // CHUNK
GROW = N_FEAT * CHUNK

_GR = TOTAL * N_FEAT // 128


def _prep_body(bidx_ref, dst_ref, pad_ref, mask_ref):
    b3 = bidx_ref[...]
    shp = (NW, N_CHUNK, CHUNK)
    i0 = lax.broadcasted_iota(jnp.int32, shp, 0)
    i1 = lax.broadcasted_iota(jnp.int32, shp, 1)
    i2 = lax.broadcasted_iota(jnp.int32, shp, 2)
    t = (i0 * N_CHUNK + i1) * CHUNK + i2

    counts = [jnp.sum((b3 == b).astype(jnp.int32)) for b in range(B)]
    starts = []
    s = jnp.int32(0)
    for b in range(B):
        starts.append(s)
        s = s + counts[b]

    ssel = jnp.zeros(shp, jnp.int32)
    for b in range(B):
        ssel = jnp.where(b3 == b, starts[b], ssel)
    dst_ref[...] = b3 * MAX_LEN + (t - ssel)

    cpad = []
    cp = jnp.int32(0)
    for b in range(B):
        cpad.append(cp)
        cp = cp + (MAX_LEN - counts[b])
    bk = jnp.zeros(shp, jnp.int32)
    for b in range(1, B):
        bk = bk + (t >= cpad[b]).astype(jnp.int32)
    csel = jnp.zeros(shp, jnp.int32)
    cpsel = jnp.zeros(shp, jnp.int32)
    for b in range(B):
        csel = jnp.where(bk == b, counts[b], csel)
        cpsel = jnp.where(bk == b, cpad[b], cpsel)
    pad_ref[...] = bk * MAX_LEN + csel + (t - cpsel)

    j = lax.broadcasted_iota(jnp.int32, (B, MAX_LEN), 1)
    row = lax.broadcasted_iota(jnp.int32, (B, MAX_LEN), 0)
    cm = jnp.zeros((B, MAX_LEN), jnp.int32)
    for b in range(B):
        cm = jnp.where(row == b, counts[b], cm)
    mask_ref[...] = j >= cm


_prep = pl.pallas_call(
    _prep_body,
    out_shape=(
        jax.ShapeDtypeStruct((NW, N_CHUNK, CHUNK), jnp.int32),
        jax.ShapeDtypeStruct((NW, N_CHUNK, CHUNK), jnp.int32),
        jax.ShapeDtypeStruct((B, MAX_LEN), jnp.bool_),
    ),
)


def _sc_body(fi_hbm, tab_hbm, dst_hbm, pad_hbm, out_hbm,
             fcol, idx0, idx1, g0, g1, acc0, acc1, zeros, dvm, pvm,
             semstage, semg0, semg1, semsc0, semsc1, semz):
    wid = lax.axis_index("s") * NC + lax.axis_index("c")

    stage = [
        pltpu.async_copy(dst_hbm.at[wid], dvm, semstage),
        pltpu.async_copy(pad_hbm.at[wid], pvm, semstage),
    ] + [
        pltpu.async_copy(fi_hbm.at[f * NW + wid], fcol.at[f], semstage)
        for f in range(N_FEAT)
    ]

    zvec = jnp.zeros((L,), jnp.float32)

    @plsc.parallel_loop(0, CHUNK, unroll=4)
    def _(t):
        for c in range(D // L):
            zeros[t, pl.ds(c * L, L)] = zvec
    for cpd in stage:
        cpd.wait()

    gs = [g0, g1]
    idxs = [idx0, idx1]
    accs = [acc0, acc1]
    semgs = [semg0, semg1]
    semscs = [semsc0, semsc1]

    def build_idx(ci):
        buf = idxs[ci % 2]
        for f in range(N_FEAT):
            for gseg in range(CHUNK // L):
                buf[pl.ds(f * CHUNK + gseg * L, L)] = (
                    fcol[f, pl.ds(ci * CHUNK + gseg * L, L)] + 1)

    def start_gather(ci):
        par = ci % 2
        return [pltpu.async_copy(
            tab_hbm.at[f].at[idxs[par].at[pl.ds(f * CHUNK, CHUNK)]],
            gs[par].at[pl.ds(f * CHUNK, CHUNK)], semgs[par])
            for f in range(N_FEAT)]

    build_idx(0)
    gcp = {0: start_gather(0)}
    sc_tok = {}
    sc_zero = []
    for ci in range(N_CHUNK):
        par = ci % 2
        if ci + 1 < N_CHUNK:
            build_idx(ci + 1)
            gcp[ci + 1] = start_gather(ci + 1)
        for cpd2 in gcp[ci]:
            cpd2.wait()
        if ci - 2 in sc_tok:
            sc_tok[ci - 2].wait()

        g = gs[par]
        acc = accs[par]

        @plsc.parallel_loop(0, CHUNK, unroll=4)
        def _(t):
            for c in range(D // L):
                co = c * L
                acc[t, pl.ds(co, L)] = (
                    g[t, pl.ds(co, L)] + g[CHUNK + t, pl.ds(co, L)]
                    + g[2 * CHUNK + t, pl.ds(co, L)]
                    + g[3 * CHUNK + t, pl.ds(co, L)])

        sc_tok[ci] = pltpu.async_copy(acc, out_hbm.at[dvm.at[ci]], semscs[par])
        sc_zero.append(pltpu.async_copy(zeros, out_hbm.at[pvm.at[ci]], semz))
    for ci in (N_CHUNK - 2, N_CHUNK - 1):
        sc_tok[ci].wait()
    for cpd in sc_zero:
        cpd.wait()


@functools.lru_cache(maxsize=None)
def _build_sc():
    mesh = plsc.VectorSubcoreMesh(
        core_axis_name="c", subcore_axis_name="s",
        num_cores=NC, num_subcores=NS)
    return pl.kernel(
        _sc_body,
        out_type=jax.ShapeDtypeStruct((B * MAX_LEN, D), jnp.float32),
        mesh=mesh,
        scratch_types=[
            pltpu.VMEM((N_FEAT, TOK_PER_W), jnp.int32),
            pltpu.VMEM((GROW,), jnp.int32),
            pltpu.VMEM((GROW,), jnp.int32),
            pltpu.VMEM((GROW, D), jnp.float32),
            pltpu.VMEM((GROW, D), jnp.float32),
            pltpu.VMEM((CHUNK, D), jnp.float32),
            pltpu.VMEM((CHUNK, D), jnp.float32),
            pltpu.VMEM((CHUNK, D), jnp.float32),
            pltpu.VMEM((N_CHUNK, CHUNK), jnp.int32),
            pltpu.VMEM((N_CHUNK, CHUNK), jnp.int32),
        ] + [pltpu.SemaphoreType.DMA for _ in range(6)],
    )


def kernel(feature_indices, batch_idx, tables):
    fi_t = jnp.transpose(feature_indices).reshape(N_FEAT * NW, TOK_PER_W)
    bidx_w = batch_idx.reshape(NW, N_CHUNK, CHUNK)
    dst_w, pad_w, mask = _prep(bidx_w)
    out2 = _build_sc()(fi_t, tables, dst_w, pad_w)
    return out2.reshape(B, MAX_LEN, D), mask


# speedup vs baseline: 1.0059x; 1.0059x over previous
import functools

import jax
import jax.numpy as jnp
from jax import lax
from jax.experimental import pallas as pl
from jax.experimental.pallas import tpu as pltpu
from jax.experimental.pallas import tpu_sc as plsc

B = 8
MAX_LEN = 2048
TOTAL = 8192
N_FEAT = 4
CARD_P2 = 1026
D = 256

NC = 2
NS = 16
L = 16
NW = NC * NS
TOK_PER_W = TOTAL // NW
CHUNK = 32
N_CHUNK = TOK_PER_W // CHUNK
GROW = N_FEAT * CHUNK

_GR = TOTAL * N_FEAT // 128


def _prep_body(bidx_ref, dst_ref, pad_ref, mask_ref):
    b3 = bidx_ref[...]
    shp = (NW, N_CHUNK, CHUNK)
    i0 = lax.broadcasted_iota(jnp.int32, shp, 0)
    i1 = lax.broadcasted_iota(jnp.int32, shp, 1)
    i2 = lax.broadcasted_iota(jnp.int32, shp, 2)
    t = (i0 * N_CHUNK + i1) * CHUNK + i2

    counts = [jnp.sum((b3 == b).astype(jnp.int32)) for b in range(B)]
    starts = []
    s = jnp.int32(0)
    for b in range(B):
        starts.append(s)
        s = s + counts[b]

    ssel = jnp.zeros(shp, jnp.int32)
    for b in range(B):
        ssel = jnp.where(b3 == b, starts[b], ssel)
    dst_ref[...] = b3 * MAX_LEN + (t - ssel)

    cpad = []
    cp = jnp.int32(0)
    for b in range(B):
        cpad.append(cp)
        cp = cp + (MAX_LEN - counts[b])
    bk = jnp.zeros(shp, jnp.int32)
    for b in range(1, B):
        bk = bk + (t >= cpad[b]).astype(jnp.int32)
    csel = jnp.zeros(shp, jnp.int32)
    cpsel = jnp.zeros(shp, jnp.int32)
    for b in range(B):
        csel = jnp.where(bk == b, counts[b], csel)
        cpsel = jnp.where(bk == b, cpad[b], cpsel)
    pad_ref[...] = bk * MAX_LEN + csel + (t - cpsel)

    j = lax.broadcasted_iota(jnp.int32, (B, MAX_LEN), 1)
    row = lax.broadcasted_iota(jnp.int32, (B, MAX_LEN), 0)
    cm = jnp.zeros((B, MAX_LEN), jnp.int32)
    for b in range(B):
        cm = jnp.where(row == b, counts[b], cm)
    mask_ref[...] = j >= cm


_prep = pl.pallas_call(
    _prep_body,
    out_shape=(
        jax.ShapeDtypeStruct((NW, N_CHUNK, CHUNK), jnp.int32),
        jax.ShapeDtypeStruct((NW, N_CHUNK, CHUNK), jnp.int32),
        jax.ShapeDtypeStruct((B, MAX_LEN), jnp.bool_),
    ),
)


def _sc_body(fi_hbm, tab_hbm, dst_hbm, pad_hbm, out_hbm,
             fcol, idx0, idx1, idx2, g0, g1, g2, acc0, acc1, zeros, dvm, pvm,
             semstage, semg0, semg1, semg2, semsc0, semsc1, semz):
    wid = lax.axis_index("s") * NC + lax.axis_index("c")

    stage = [
        pltpu.async_copy(dst_hbm.at[wid], dvm, semstage),
        pltpu.async_copy(pad_hbm.at[wid], pvm, semstage),
    ] + [
        pltpu.async_copy(fi_hbm.at[f * NW + wid], fcol.at[f], semstage)
        for f in range(N_FEAT)
    ]

    zvec = jnp.zeros((L,), jnp.float32)

    @plsc.parallel_loop(0, CHUNK, unroll=4)
    def _(t):
        for c in range(D // L):
            zeros[t, pl.ds(c * L, L)] = zvec
    for cpd in stage:
        cpd.wait()

    gs = [g0, g1, g2]
    idxs = [idx0, idx1, idx2]
    accs = [acc0, acc1]
    semgs = [semg0, semg1, semg2]
    semscs = [semsc0, semsc1]

    def build_idx(ci):
        buf = idxs[ci % 3]
        for f in range(N_FEAT):
            for gseg in range(CHUNK // L):
                buf[pl.ds(f * CHUNK + gseg * L, L)] = (
                    fcol[f, pl.ds(ci * CHUNK + gseg * L, L)] + 1)

    def start_gather(ci):
        par = ci % 3
        return [pltpu.async_copy(
            tab_hbm.at[f].at[idxs[par].at[pl.ds(f * CHUNK, CHUNK)]],
            gs[par].at[pl.ds(f * CHUNK, CHUNK)], semgs[par])
            for f in range(N_FEAT)]

    build_idx(0)
    gcp = {0: start_gather(0)}
    if N_CHUNK > 1:
        build_idx(1)
        gcp[1] = start_gather(1)
    sc_tok = {}
    sc_zero = []
    for ci in range(N_CHUNK):
        par = ci % 3
        apar = ci % 2
        if ci + 2 < N_CHUNK:
            build_idx(ci + 2)
            gcp[ci + 2] = start_gather(ci + 2)
        for cpd2 in gcp[ci]:
            cpd2.wait()
        if ci - 2 in sc_tok:
            sc_tok[ci - 2].wait()

        g = gs[par]
        acc = accs[apar]

        @plsc.parallel_loop(0, CHUNK, unroll=4)
        def _(t):
            for c in range(D // L):
                co = c * L
                acc[t, pl.ds(co, L)] = (
                    g[t, pl.ds(co, L)] + g[CHUNK + t, pl.ds(co, L)]
                    + g[2 * CHUNK + t, pl.ds(co, L)]
                    + g[3 * CHUNK + t, pl.ds(co, L)])

        sc_tok[ci] = pltpu.async_copy(acc, out_hbm.at[dvm.at[ci]], semscs[apar])
        sc_zero.append(pltpu.async_copy(zeros, out_hbm.at[pvm.at[ci]], semz))
    for ci in (N_CHUNK - 2, N_CHUNK - 1):
        sc_tok[ci].wait()
    for cpd in sc_zero:
        cpd.wait()


@functools.lru_cache(maxsize=None)
def _build_sc():
    mesh = plsc.VectorSubcoreMesh(
        core_axis_name="c", subcore_axis_name="s",
        num_cores=NC, num_subcores=NS)
    return pl.kernel(
        _sc_body,
        out_type=jax.ShapeDtypeStruct((B * MAX_LEN, D), jnp.float32),
        mesh=mesh,
        scratch_types=[
            pltpu.VMEM((N_FEAT, TOK_PER_W), jnp.int32),
            pltpu.VMEM((GROW,), jnp.int32),
            pltpu.VMEM((GROW,), jnp.int32),
            pltpu.VMEM((GROW,), jnp.int32),
            pltpu.VMEM((GROW, D), jnp.float32),
            pltpu.VMEM((GROW, D), jnp.float32),
            pltpu.VMEM((GROW, D), jnp.float32),
            pltpu.VMEM((CHUNK, D), jnp.float32),
            pltpu.VMEM((CHUNK, D), jnp.float32),
            pltpu.VMEM((CHUNK, D), jnp.float32),
            pltpu.VMEM((N_CHUNK, CHUNK), jnp.int32),
            pltpu.VMEM((N_CHUNK, CHUNK), jnp.int32),
        ] + [pltpu.SemaphoreType.DMA for _ in range(7)],
    )


def kernel(feature_indices, batch_idx, tables):
    fi_t = jnp.transpose(feature_indices).reshape(N_FEAT * NW, TOK_PER_W)
    bidx_w = batch_idx.reshape(NW, N_CHUNK, CHUNK)
    dst_w, pad_w, mask = _prep(bidx_w)
    out2 = _build_sc()(fi_t, tables, dst_w, pad_w)
    return out2.reshape(B, MAX_LEN, D), mask
